# baseline (speedup 1.0000x reference)
import jax
import jax.numpy as jnp
from jax.experimental import pallas as pl
from jax.experimental.pallas import tpu as pltpu

_CHUNK = 256000
_NSTEPS = 25


def _body(k_ref, b_hbm, o_ref, scratch, sems):
    i = pl.program_id(0)
    k = k_ref[0]

    @pl.when(i == 0)
    def _enqueue_all():
        for j in range(_NSTEPS):
            pltpu.make_async_copy(
                b_hbm.at[k, pl.ds(j * _CHUNK, _CHUNK)],
                scratch.at[pl.ds(j * _CHUNK, _CHUNK)], sems.at[j],
            ).start()

    pltpu.make_async_copy(
        b_hbm.at[k, pl.ds(i * _CHUNK, _CHUNK)],
        scratch.at[pl.ds(i * _CHUNK, _CHUNK)], sems.at[i],
    ).wait()
    o_ref[...] = jax.nn.sigmoid(scratch[pl.ds(i * _CHUNK, _CHUNK)])


def kernel(B, k, edge_index, n):
    E = B.shape[1]
    k_arr = jnp.atleast_1d(k).astype(jnp.int32)
    out = pl.pallas_call(
        _body,
        grid_spec=pltpu.PrefetchScalarGridSpec(
            num_scalar_prefetch=1,
            grid=(_NSTEPS,),
            in_specs=[pl.BlockSpec(memory_space=pl.ANY)],
            out_specs=pl.BlockSpec((_CHUNK,), lambda i, kref: (i,)),
            scratch_shapes=[
                pltpu.VMEM((_NSTEPS * _CHUNK,), jnp.float32),
                pltpu.SemaphoreType.DMA((_NSTEPS,)),
            ],
        ),
        out_shape=jax.ShapeDtypeStruct((E,), jnp.float32),
    )(k_arr, B)
    return out


# speedup vs baseline: 1.1692x; 1.1692x over previous
import jax
import jax.numpy as jnp
from jax.experimental import pallas as pl
from jax.experimental.pallas import tpu as pltpu

_CHUNK = 1280000
_NSTEPS = 5


def _body(k_ref, b_hbm, o_ref, scratch, sems):
    i = pl.program_id(0)
    k = k_ref[0]

    @pl.when(i == 0)
    def _enqueue_all():
        for j in range(_NSTEPS):
            pltpu.make_async_copy(
                b_hbm.at[k, pl.ds(j * _CHUNK, _CHUNK)],
                scratch.at[pl.ds(j * _CHUNK, _CHUNK)], sems.at[j],
            ).start()

    pltpu.make_async_copy(
        b_hbm.at[k, pl.ds(i * _CHUNK, _CHUNK)],
        scratch.at[pl.ds(i * _CHUNK, _CHUNK)], sems.at[i],
    ).wait()
    o_ref[...] = jax.nn.sigmoid(scratch[pl.ds(i * _CHUNK, _CHUNK)])


def kernel(B, k, edge_index, n):
    E = B.shape[1]
    k_arr = jnp.atleast_1d(k).astype(jnp.int32)
    out = pl.pallas_call(
        _body,
        grid_spec=pltpu.PrefetchScalarGridSpec(
            num_scalar_prefetch=1,
            grid=(_NSTEPS,),
            in_specs=[pl.BlockSpec(memory_space=pl.ANY)],
            out_specs=pl.BlockSpec((_CHUNK,), lambda i, kref: (i,)),
            scratch_shapes=[
                pltpu.VMEM((_NSTEPS * _CHUNK,), jnp.float32),
                pltpu.SemaphoreType.DMA((_NSTEPS,)),
            ],
        ),
        out_shape=jax.ShapeDtypeStruct((E,), jnp.float32),
    )(k_arr, B)
    return out
